# baseline (speedup 1.0000x reference)
import functools

import jax
import jax.numpy as jnp
from jax import lax
from jax.experimental import pallas as pl
from jax.experimental.pallas import tpu as pltpu
from jax.experimental.pallas import tpu_sc as plsc

N_ATTRIBUTES = 26
N_VALUES = 128
LEVEL = 13
D = N_ATTRIBUTES * N_VALUES
KEEP = LEVEL * N_VALUES
ZERO = D - KEEP
BM = 1024


def _mask_kernel(x_ref, out_ref):
    out_ref[:, :KEEP] = x_ref[...]
    out_ref[:, KEEP:] = jnp.zeros((x_ref.shape[0], ZERO), x_ref.dtype)


def kernel(sender_input, labels):
    B = sender_input.shape[0]
    grid = (B // BM,)
    masked = pl.pallas_call(
        _mask_kernel,
        grid=grid,
        in_specs=[pl.BlockSpec((BM, KEEP), lambda i: (i, 0))],
        out_specs=pl.BlockSpec((BM, D), lambda i: (i, 0)),
        out_shape=jax.ShapeDtypeStruct((B, D), sender_input.dtype),
    )(sender_input)
    idx_flat, n_revealed = _aux_sc_kernel(B)
    return masked, idx_flat.reshape(B, LEVEL), n_revealed


def _aux_sc_kernel(B):
    NC, NS = 2, 16
    NW = NC * NS
    idx_w = B * LEVEL // NW
    nrev_w = B // NW
    period = 16 * LEVEL

    mesh = plsc.VectorSubcoreMesh(core_axis_name="c", subcore_axis_name="s")

    @functools.partial(
        pl.kernel, mesh=mesh,
        out_type=[
            jax.ShapeDtypeStruct((B * LEVEL,), jnp.int32),
            jax.ShapeDtypeStruct((B,), jnp.int32),
        ],
        scratch_types=[
            pltpu.VMEM((idx_w,), jnp.int32),
            pltpu.VMEM((nrev_w,), jnp.int32),
        ],
    )
    def aux(idx_hbm, nrev_hbm, idx_v, nrev_v):
        wid = lax.axis_index("s") * NC + lax.axis_index("c")
        lanes = lax.iota(jnp.int32, 16)
        pats = [(lanes + jnp.int32(16 * k)) % jnp.int32(LEVEL) for k in range(LEVEL)]
        for m in range(idx_w // 16):
            idx_v[pl.ds(16 * m, 16)] = pats[m % LEVEL]
        thirteen = jnp.full((16,), LEVEL, jnp.int32)
        for m in range(nrev_w // 16):
            nrev_v[pl.ds(16 * m, 16)] = thirteen
        pltpu.sync_copy(idx_v, idx_hbm.at[pl.ds(wid * idx_w, idx_w)])
        pltpu.sync_copy(nrev_v, nrev_hbm.at[pl.ds(wid * nrev_w, nrev_w)])

    return aux()


# speedup vs baseline: 1.1335x; 1.1335x over previous
import jax
import jax.numpy as jnp
from jax.experimental import pallas as pl

N_ATTRIBUTES = 26
N_VALUES = 128
LEVEL = 13
D = N_ATTRIBUTES * N_VALUES
KEEP = LEVEL * N_VALUES
ZERO = D - KEEP
BM = 1024


def _mask_kernel(x_ref, out_ref, idx_ref, nrev_ref):
    out_ref[:, :KEEP] = x_ref[...]
    out_ref[:, KEEP:] = jnp.zeros((x_ref.shape[0], ZERO), x_ref.dtype)

    @pl.when(pl.program_id(0) == 0)
    def _aux():
        idx_ref[...] = jax.lax.broadcasted_iota(jnp.int32, idx_ref.shape, 1)
        nrev_ref[...] = jnp.full(nrev_ref.shape, LEVEL, jnp.int32)


def kernel(sender_input, labels):
    B = sender_input.shape[0]
    grid = (B // BM,)
    masked, idxs_to_reveal, n_revealed = pl.pallas_call(
        _mask_kernel,
        grid=grid,
        in_specs=[pl.BlockSpec((BM, KEEP), lambda i: (i, 0))],
        out_specs=[
            pl.BlockSpec((BM, D), lambda i: (i, 0)),
            pl.BlockSpec((B, LEVEL), lambda i: (0, 0)),
            pl.BlockSpec((B, 1), lambda i: (0, 0)),
        ],
        out_shape=[
            jax.ShapeDtypeStruct((B, D), sender_input.dtype),
            jax.ShapeDtypeStruct((B, LEVEL), jnp.int32),
            jax.ShapeDtypeStruct((B, 1), jnp.int32),
        ],
    )(sender_input)
    return masked, idxs_to_reveal, n_revealed.reshape(B)


# speedup vs baseline: 1.3043x; 1.1507x over previous
import jax
import jax.numpy as jnp
from jax.experimental import pallas as pl

N_ATTRIBUTES = 26
N_VALUES = 128
LEVEL = 13
D = N_ATTRIBUTES * N_VALUES
KEEP = LEVEL * N_VALUES
ZERO = D - KEEP
BM = 1024


def _mask_kernel(x_ref, out_ref):
    out_ref[:, :KEEP] = x_ref[...]
    out_ref[:, KEEP:] = jnp.zeros((x_ref.shape[0], ZERO), x_ref.dtype)


def kernel(sender_input, labels):
    B = sender_input.shape[0]
    grid = (B // BM,)
    masked = pl.pallas_call(
        _mask_kernel,
        grid=grid,
        in_specs=[pl.BlockSpec((BM, KEEP), lambda i: (i, 0))],
        out_specs=pl.BlockSpec((BM, D), lambda i: (i, 0)),
        out_shape=jax.ShapeDtypeStruct((B, D), sender_input.dtype),
    )(sender_input)
    idxs_to_reveal = jnp.broadcast_to(
        jnp.arange(LEVEL, dtype=jnp.int32), (B, LEVEL)
    )
    n_revealed = jnp.full((B,), LEVEL, dtype=jnp.int32)
    return masked, idxs_to_reveal, n_revealed
